# baseline (speedup 1.0000x reference)
import functools

import jax
import jax.numpy as jnp
from jax import lax
from jax.experimental import pallas as pl
from jax.experimental.pallas import tpu as pltpu
from jax.experimental.pallas import tpu_sc as plsc

_DIM = 64
_C = 128
_SUPI = 512


@functools.partial(jax.jit, static_argnames=("total",))
def _lookup(x_flat, table, total):
    info = plsc.get_sparse_core_info()
    nw = info.num_cores * info.num_subcores
    b_per_w = total // nw
    n_sup = b_per_w // _SUPI
    n_chunks_per_sup = _SUPI // _C
    mesh = plsc.VectorSubcoreMesh(core_axis_name="c", subcore_axis_name="s")

    @functools.partial(
        pl.kernel,
        mesh=mesh,
        compiler_params=pltpu.CompilerParams(needs_layout_passes=False),
        out_type=jax.ShapeDtypeStruct((total, _DIM), jnp.float32),
        scratch_types=[
            pltpu.VMEM((128, _DIM), jnp.float32),
            pltpu.VMEM((128 * _DIM,), jnp.float32),
            pltpu.VMEM((_SUPI,), jnp.int32),
            pltpu.VMEM((2 * _C, _DIM), jnp.float32),
            pltpu.SemaphoreType.DMA((2,)),
        ],
    )
    def k(x_hbm, table_hbm, out_hbm, tab_v, tab_flat, idx_v, rows_v, wsem):
        wid = lax.axis_index("s") * info.num_cores + lax.axis_index("c")
        base = wid * b_per_w
        lanes = lax.iota(jnp.int32, 16)

        pltpu.sync_copy(table_hbm, tab_v)

        def flat_body(v, _):
            for c in range(_DIM // 16):
                tab_flat[pl.ds(v * _DIM + c * 16, 16)] = tab_v[
                    v, pl.ds(c * 16, 16)
                ]
            return 0

        lax.fori_loop(0, 128, flat_body, 0)

        def sup_body(s, _):
            off = base + s * _SUPI
            pltpu.sync_copy(x_hbm.at[pl.ds(off, _SUPI)], idx_v)

            def chunk_body(g, _):
                i_glob = s * n_chunks_per_sup + g
                buf = lax.rem(i_glob, 2)

                @pl.when(i_glob >= 2)
                def _wait_prev():
                    pltpu.make_async_copy(
                        rows_v.at[pl.ds(buf * _C, _C)],
                        out_hbm.at[pl.ds(0, _C)],
                        wsem.at[buf],
                    ).wait()

                @plsc.parallel_loop(0, _C // 16, unroll=2)
                def grp_body(q):
                    ivec = idx_v[pl.ds(g * _C + q * 16, 16)]
                    avec = ivec * _DIM
                    rvec = buf * _C + q * 16 + lanes
                    for c0 in range(0, _DIM, 16):
                        cols = [
                            (lanes + c) & (_DIM - 1)
                            for c in range(c0, c0 + 16)
                        ]
                        vals = [
                            plsc.load_gather(tab_flat, [avec + colv])
                            for colv in cols
                        ]
                        for colv, v in zip(cols, vals):
                            plsc.store_scatter(rows_v, [rvec, colv], v)
                pltpu.async_copy(
                    rows_v.at[pl.ds(buf * _C, _C)],
                    out_hbm.at[pl.ds(off + g * _C, _C)],
                    wsem.at[buf],
                )
                return 0

            lax.fori_loop(0, n_chunks_per_sup, chunk_body, 0)
            return 0

        lax.fori_loop(0, n_sup, sup_body, 0)

        for b in range(2):
            pltpu.make_async_copy(
                rows_v.at[pl.ds(b * _C, _C)],
                out_hbm.at[pl.ds(0, _C)],
                wsem.at[b],
            ).wait()

    return k(x_flat, table)


def kernel(x, table):
    total = x.shape[0] * x.shape[1]
    x_flat = jnp.ravel(x).astype(jnp.int32)
    out = _lookup(x_flat, table, total)
    return out.reshape(x.shape[0], x.shape[1], _DIM)


# speedup vs baseline: 2.4421x; 2.4421x over previous
import functools

import jax
import jax.numpy as jnp
from jax import lax
from jax.experimental import pallas as pl
from jax.experimental.pallas import tpu as pltpu
from jax.experimental.pallas import tpu_sc as plsc

_DIM = 64
_C = 256
_SUPI = 4096


@functools.partial(jax.jit, static_argnames=("total",))
def _lookup(x_flat, table, total):
    info = plsc.get_sparse_core_info()
    nw = info.num_cores * info.num_subcores
    b_per_w = total // nw
    n_sup = b_per_w // _SUPI
    n_chunks_per_sup = _SUPI // _C
    mesh = plsc.VectorSubcoreMesh(core_axis_name="c", subcore_axis_name="s")

    @functools.partial(
        pl.kernel,
        mesh=mesh,
        compiler_params=pltpu.CompilerParams(needs_layout_passes=False),
        out_type=jax.ShapeDtypeStruct((total, _DIM), jnp.float32),
        scratch_types=[
            pltpu.VMEM((128, _DIM), jnp.float32),
            pltpu.VMEM((128 * _DIM,), jnp.float32),
            pltpu.VMEM((_SUPI,), jnp.int32),
            pltpu.VMEM((2 * _C, _DIM), jnp.float32),
            pltpu.SemaphoreType.DMA((2,)),
        ],
    )
    def k(x_hbm, table_hbm, out_hbm, tab_v, tab_flat, idx_v, rows_v, wsem):
        wid = lax.axis_index("s") * info.num_cores + lax.axis_index("c")
        base = wid * b_per_w
        lanes = lax.iota(jnp.int32, 16)

        pltpu.sync_copy(table_hbm, tab_v)

        def flat_body(v, _):
            for c in range(_DIM // 16):
                tab_flat[pl.ds(v * _DIM + c * 16, 16)] = tab_v[
                    v, pl.ds(c * 16, 16)
                ]
            return 0

        lax.fori_loop(0, 128, flat_body, 0)

        def sup_body(s, _):
            off = base + s * _SUPI
            pltpu.sync_copy(x_hbm.at[pl.ds(off, _SUPI)], idx_v)

            def chunk_body(g, _):
                i_glob = s * n_chunks_per_sup + g
                buf = lax.rem(i_glob, 2)

                @pl.when(i_glob >= 2)
                def _wait_prev():
                    pltpu.make_async_copy(
                        rows_v.at[pl.ds(buf * _C, _C)],
                        out_hbm.at[pl.ds(0, _C)],
                        wsem.at[buf],
                    ).wait()

                @plsc.parallel_loop(0, _C // 16)
                def grp_body(q):
                    ivec = idx_v[pl.ds(g * _C + q * 16, 16)]
                    avec = ivec * _DIM
                    rvec = buf * _C + q * 16 + lanes
                    for c0 in range(0, _DIM, 16):
                        cols = [
                            (lanes + c) & (_DIM - 1)
                            for c in range(c0, c0 + 16)
                        ]
                        vals = [
                            plsc.load_gather(tab_flat, [avec + colv])
                            for colv in cols
                        ]
                        for colv, v in zip(cols, vals):
                            plsc.store_scatter(rows_v, [rvec, colv], v)
                pltpu.async_copy(
                    rows_v.at[pl.ds(buf * _C, _C)],
                    out_hbm.at[pl.ds(off + g * _C, _C)],
                    wsem.at[buf],
                )
                return 0

            lax.fori_loop(0, n_chunks_per_sup, chunk_body, 0)
            return 0

        lax.fori_loop(0, n_sup, sup_body, 0)

        for b in range(2):
            pltpu.make_async_copy(
                rows_v.at[pl.ds(b * _C, _C)],
                out_hbm.at[pl.ds(0, _C)],
                wsem.at[b],
            ).wait()

    return k(x_flat, table)


def kernel(x, table):
    total = x.shape[0] * x.shape[1]
    x_flat = jnp.ravel(x).astype(jnp.int32)
    out = _lookup(x_flat, table, total)
    return out.reshape(x.shape[0], x.shape[1], _DIM)
